# Initial kernel scaffold; baseline (speedup 1.0000x reference)
import jax
import jax.numpy as jnp
from jax.experimental import pallas as pl


def kernel(z, pos_edge_index, pos_edge_weights, neg_edge_index):
    raise NotImplementedError("write your pallas kernel here")


# speedup vs baseline: 2.3100x; 2.3100x over previous
import jax
import jax.numpy as jnp
from jax import lax
from jax.experimental import pallas as pl
from jax.experimental.pallas import tpu as pltpu
from jax.experimental.pallas import tpu_sc as plsc

NC = 2
NS = 16
L = 16
NW = NC * NS
D = 128
E = 320000
EPW = 2 * E // NW
CHUNK = 80
NCHUNK = EPW // CHUNK
GRPS = CHUNK // L
KV = D // L
EPS = 1e-15
LN2 = 0.6931471805599453


def _vlog(x):
    bits = plsc.bitcast(x, jnp.int32)
    e = (bits >> 23) - 127
    m = plsc.bitcast((bits & 0x7FFFFF) | 0x3F800000, jnp.float32)
    big = m >= jnp.float32(1.4142135)
    m = jnp.where(big, m * jnp.float32(0.5), m)
    ef = (e + big.astype(jnp.int32)).astype(jnp.float32)
    s = (m - jnp.float32(1.0)) / (m + jnp.float32(1.0))
    u = s * s
    p = jnp.float32(1.0 / 11.0)
    for c in (1.0 / 9.0, 1.0 / 7.0, 1.0 / 5.0, 1.0 / 3.0):
        p = p * u + jnp.float32(c)
    return ef * LN2 + jnp.float32(2.0) * s * (jnp.float32(1.0) + u * p)


def _body(z_hbm, src_hbm, dst_hbm, out_hbm,
          idx_s, idx_d, rows_s, rows_d, scr, accv, redv, outv, shared,
          sem_s, sem_d):
    cid = lax.axis_index("c")
    sid = lax.axis_index("s")
    wid = sid * NC + cid
    base_w = wid * EPW
    negv = jnp.full((L,), wid >= NW // 2)

    accv[...] = jnp.zeros((L,), jnp.float32)

    def chunk_body(c, carry):
        base = base_w + c * CHUNK
        pltpu.sync_copy(src_hbm.at[pl.ds(base, CHUNK)], idx_s)
        pltpu.sync_copy(dst_hbm.at[pl.ds(base, CHUNK)], idx_d)
        cp_s = pltpu.async_copy(z_hbm.at[idx_s], rows_s, sem_s)
        cp_d = pltpu.async_copy(z_hbm.at[idx_d], rows_d, sem_d)
        cp_s.wait()
        cp_d.wait()
        iota = lax.iota(jnp.int32, L)
        for g in range(GRPS):
            for e_ in range(L):
                r = g * L + e_
                a = rows_s[r, pl.ds(0, L)] * rows_d[r, pl.ds(0, L)]
                for k in range(1, KV):
                    a = a + rows_s[r, pl.ds(k * L, L)] * rows_d[r, pl.ds(k * L, L)]
                scr[pl.ds(e_ * L, L)] = a
            t = plsc.load_gather(scr, [iota * L])
            for l in range(1, L):
                t = t + plsc.load_gather(scr, [iota * L + l])
            prob = jnp.float32(1.0) / (jnp.float32(1.0) + jnp.exp(-t))
            arg = jnp.where(negv, (jnp.float32(1.0) - prob) + EPS, prob + EPS)
            accv[...] = accv[...] - _vlog(arg)
        return carry

    lax.fori_loop(0, NCHUNK, chunk_body, jnp.int32(0))

    pltpu.sync_copy(accv, shared.at[sid])
    plsc.subcore_barrier()

    @pl.when(sid == 0)
    def _():
        pltpu.sync_copy(shared, redv)
        tot = redv[0, :]
        for s_ in range(1, NS):
            tot = tot + redv[s_, :]
        total = jnp.sum(tot) * jnp.float32(1.0 / E)
        outv[...] = jnp.full((L,), total, jnp.float32)
        pltpu.sync_copy(outv, out_hbm.at[cid])


_mesh = plsc.VectorSubcoreMesh(
    core_axis_name="c", subcore_axis_name="s", num_cores=NC, num_subcores=NS)

_sc_call = pl.kernel(
    _body,
    out_type=jax.ShapeDtypeStruct((NC, L), jnp.float32),
    mesh=_mesh,
    scratch_types=[
        pltpu.VMEM((CHUNK,), jnp.int32),
        pltpu.VMEM((CHUNK,), jnp.int32),
        pltpu.VMEM((CHUNK, D), jnp.float32),
        pltpu.VMEM((CHUNK, D), jnp.float32),
        pltpu.VMEM((L * L,), jnp.float32),
        pltpu.VMEM((L,), jnp.float32),
        pltpu.VMEM((NS, L), jnp.float32),
        pltpu.VMEM((L,), jnp.float32),
        pltpu.VMEM_SHARED((NS, L), jnp.float32),
        pltpu.SemaphoreType.DMA,
        pltpu.SemaphoreType.DMA,
    ],
    compiler_params=pltpu.CompilerParams(needs_layout_passes=False),
)


@jax.jit
def kernel(z, pos_edge_index, pos_edge_weights, neg_edge_index):
    del pos_edge_weights
    src = jnp.concatenate(
        [pos_edge_index[0], neg_edge_index[0]]).astype(jnp.int32)
    dst = jnp.concatenate(
        [pos_edge_index[1], neg_edge_index[1]]).astype(jnp.int32)
    out = _sc_call(z, src, dst)
    return out[0, 0] + out[1, 0]


# speedup vs baseline: 3.4201x; 1.4806x over previous
import jax
import jax.numpy as jnp
from jax import lax
from jax.experimental import pallas as pl
from jax.experimental.pallas import tpu as pltpu
from jax.experimental.pallas import tpu_sc as plsc

NC = 2
NS = 16
L = 16
NW = NC * NS
D = 128
E = 320000
EPW = 2 * E // NW
CHUNK = 80
NCHUNK = EPW // CHUNK
NPAIR = NCHUNK // 2
GRPS = CHUNK // L
KV = D // L
EPS = 1e-15
LN2 = 0.6931471805599453


def _vlog(x):
    bits = plsc.bitcast(x, jnp.int32)
    e = (bits >> 23) - 127
    m = plsc.bitcast((bits & 0x7FFFFF) | 0x3F800000, jnp.float32)
    big = m >= jnp.float32(1.4142135)
    m = jnp.where(big, m * jnp.float32(0.5), m)
    ef = (e + big.astype(jnp.int32)).astype(jnp.float32)
    s = (m - jnp.float32(1.0)) / (m + jnp.float32(1.0))
    u = s * s
    p = jnp.float32(1.0 / 11.0)
    for c in (1.0 / 9.0, 1.0 / 7.0, 1.0 / 5.0, 1.0 / 3.0):
        p = p * u + jnp.float32(c)
    return ef * jnp.float32(LN2) + jnp.float32(2.0) * s * (jnp.float32(1.0) + u * p)


def _body(z_hbm, src_hbm, dst_hbm, out_hbm,
          idx_s, idx_d, rows_s0, rows_d0, rows_s1, rows_d1,
          scr, accv, redv, outv, shared, sem0, sem1):
    cid = lax.axis_index("c")
    sid = lax.axis_index("s")
    wid = sid * NC + cid
    base_w = wid * EPW
    negv = jnp.full((L,), wid >= NW // 2)
    iota = lax.iota(jnp.int32, L)

    pltpu.sync_copy(src_hbm.at[pl.ds(base_w, EPW)], idx_s)
    pltpu.sync_copy(dst_hbm.at[pl.ds(base_w, EPW)], idx_d)

    def fire(c, rows_s, rows_d, sem):
        off = c * CHUNK
        pltpu.async_copy(z_hbm.at[idx_s.at[pl.ds(off, CHUNK)]], rows_s, sem)
        pltpu.async_copy(z_hbm.at[idx_d.at[pl.ds(off, CHUNK)]], rows_d, sem)

    def drain(rows_s, rows_d, sem):
        pltpu.make_async_copy(z_hbm.at[pl.ds(0, CHUNK)], rows_s, sem).wait()
        pltpu.make_async_copy(z_hbm.at[pl.ds(0, CHUNK)], rows_d, sem).wait()

    def compute(rows_s, rows_d, acc):
        for g in range(GRPS):
            for e_ in range(L):
                r = g * L + e_
                a = rows_s[r, pl.ds(0, L)] * rows_d[r, pl.ds(0, L)]
                for k in range(1, KV):
                    a = a + rows_s[r, pl.ds(k * L, L)] * rows_d[r, pl.ds(k * L, L)]
                scr[pl.ds(e_ * L, L)] = a
            t = plsc.load_gather(scr, [iota * L])
            for l in range(1, L):
                t = t + plsc.load_gather(scr, [iota * L + l])
            prob = jnp.float32(1.0) / (jnp.float32(1.0) + jnp.exp(-t))
            arg = jnp.where(negv,
                            (jnp.float32(1.0) - prob) + jnp.float32(EPS),
                            prob + jnp.float32(EPS))
            acc = acc - _vlog(arg)
        return acc

    fire(0, rows_s0, rows_d0, sem0)

    def pair_body(i, acc):
        fire(2 * i + 1, rows_s1, rows_d1, sem1)
        drain(rows_s0, rows_d0, sem0)
        acc = compute(rows_s0, rows_d0, acc)

        @pl.when(i < NPAIR - 1)
        def _():
            fire(2 * i + 2, rows_s0, rows_d0, sem0)

        drain(rows_s1, rows_d1, sem1)
        acc = compute(rows_s1, rows_d1, acc)
        return acc

    acc = lax.fori_loop(0, NPAIR, pair_body, jnp.zeros((L,), jnp.float32))
    accv[...] = acc

    pltpu.sync_copy(accv, shared.at[sid])
    plsc.subcore_barrier()

    @pl.when(sid == 0)
    def _():
        pltpu.sync_copy(shared, redv)
        tot = redv[0, :]
        for s_ in range(1, NS):
            tot = tot + redv[s_, :]
        total = jnp.sum(tot) * jnp.float32(1.0 / E)
        outv[...] = jnp.full((L,), total, jnp.float32)
        pltpu.sync_copy(outv, out_hbm.at[cid])


_mesh = plsc.VectorSubcoreMesh(
    core_axis_name="c", subcore_axis_name="s", num_cores=NC, num_subcores=NS)

_sc_call = pl.kernel(
    _body,
    out_type=jax.ShapeDtypeStruct((NC, L), jnp.float32),
    mesh=_mesh,
    scratch_types=[
        pltpu.VMEM((EPW,), jnp.int32),
        pltpu.VMEM((EPW,), jnp.int32),
        pltpu.VMEM((CHUNK, D), jnp.float32),
        pltpu.VMEM((CHUNK, D), jnp.float32),
        pltpu.VMEM((CHUNK, D), jnp.float32),
        pltpu.VMEM((CHUNK, D), jnp.float32),
        pltpu.VMEM((L * L,), jnp.float32),
        pltpu.VMEM((L,), jnp.float32),
        pltpu.VMEM((NS, L), jnp.float32),
        pltpu.VMEM((L,), jnp.float32),
        pltpu.VMEM_SHARED((NS, L), jnp.float32),
        pltpu.SemaphoreType.DMA,
        pltpu.SemaphoreType.DMA,
    ],
    compiler_params=pltpu.CompilerParams(needs_layout_passes=False),
)


@jax.jit
def kernel(z, pos_edge_index, pos_edge_weights, neg_edge_index):
    del pos_edge_weights
    src = jnp.concatenate(
        [pos_edge_index[0], neg_edge_index[0]]).astype(jnp.int32)
    dst = jnp.concatenate(
        [pos_edge_index[1], neg_edge_index[1]]).astype(jnp.int32)
    out = _sc_call(z, src, dst)
    return out[0, 0] + out[1, 0]
